# Initial kernel scaffold; baseline (speedup 1.0000x reference)
import jax
import jax.numpy as jnp
from jax.experimental import pallas as pl


def kernel(input_ids, labels, emb_table, W1, b1, W2, b2):
    raise NotImplementedError("write your pallas kernel here")


# speedup vs baseline: 15.9959x; 15.9959x over previous
import functools

import jax
import jax.numpy as jnp
from jax import lax
from jax.experimental import pallas as pl
from jax.experimental.pallas import tpu as pltpu
from jax.experimental.pallas import tpu_sc as plsc

VOCAB = 100000
EMB = 128
B = 4096
L = 200
NUM_CLASSES = 2
HIDDEN = 200

NC = 2
NS = 16
NW = NC * NS
ROWS_W = B // NW
CHUNK = L // 2
CHUNKS_W = ROWS_W * 2
NBUF = 4


def _sc_body(table_hbm, ids_hbm, out_hbm, idx_v, bufs, pooled_v, sems):
    cid = lax.axis_index("c")
    sid = lax.axis_index("s")
    wid = sid * NC + cid
    rbase = wid * CHUNKS_W
    obase = wid * ROWS_W

    pltpu.sync_copy(ids_hbm.at[pl.ds(rbase, CHUNKS_W)], idx_v)

    def gather(c, k):
        return pltpu.make_async_copy(table_hbm.at[idx_v.at[c]], bufs.at[k],
                                     sems.at[k])

    for k in range(NBUF):
        gather(k, k).start()

    def outer(i, _):
        for half in range(2):
            acc = tuple(jnp.zeros((16,), jnp.float32) for _ in range(8))
            for k2 in range(2):
                k = half * 2 + k2
                c = 4 * i + k
                gather(c, k).wait()

                def red(j, carry, k=k):
                    return tuple(carry[d] + bufs[k, j, pl.ds(d * 16, 16)]
                                 for d in range(8))

                acc = lax.fori_loop(0, CHUNK, red, acc)

                @pl.when(c + NBUF < CHUNKS_W)
                def _(c=c, k=k):
                    gather(c + NBUF, k).start()

            row = 2 * i + half
            inv_l = jnp.float32(1.0 / L)
            for d in range(8):
                pooled_v[row, pl.ds(d * 16, 16)] = acc[d] * inv_l
        return 0

    lax.fori_loop(0, ROWS_W // 2, outer, 0)
    pltpu.sync_copy(pooled_v, out_hbm.at[pl.ds(obase, ROWS_W)])


@functools.partial(jax.jit, static_argnums=())
def _sc_pool(emb_table, ids2):
    mesh = plsc.VectorSubcoreMesh(core_axis_name="c", subcore_axis_name="s",
                                  num_cores=NC, num_subcores=NS)
    return pl.kernel(
        _sc_body,
        out_type=jax.ShapeDtypeStruct((B, EMB), jnp.float32),
        mesh=mesh,
        scratch_types=[
            pltpu.VMEM((CHUNKS_W, CHUNK), jnp.int32),
            pltpu.VMEM((NBUF, CHUNK, EMB), jnp.float32),
            pltpu.VMEM((ROWS_W, EMB), jnp.float32),
            pltpu.SemaphoreType.DMA((NBUF,)),
        ],
    )(emb_table, ids2)


def _tc_body(x_ref, w1_ref, b1_ref, w2_ref, b2_ref, lab_ref,
             logits_ref, loss_ref):
    x = x_ref[...]
    h = jnp.maximum(
        jnp.dot(x, w1_ref[...], preferred_element_type=jnp.float32)
        + b1_ref[...], 0.0)
    lg = (jnp.dot(h, w2_ref[...], preferred_element_type=jnp.float32)
          + b2_ref[...])
    logits_ref[...] = lg
    col = lax.broadcasted_iota(jnp.int32, (B, EMB), 1)
    valid = col < NUM_CLASSES
    m = jnp.max(jnp.where(valid, lg, jnp.float32(-1e30)), axis=1,
                keepdims=True)
    se = jnp.sum(jnp.where(valid, jnp.exp(lg - m), 0.0), axis=1,
                 keepdims=True)
    lse = m + jnp.log(se)
    picked = jnp.sum(jnp.where(col == lab_ref[...], lg, 0.0), axis=1,
                     keepdims=True)
    loss_ref[0, 0] = jnp.sum(lse - picked) / jnp.float32(B)


def _tc_mlp(pooled, W1, b1, W2p, b2p, labels2d):
    return pl.pallas_call(
        _tc_body,
        out_shape=(
            jax.ShapeDtypeStruct((B, EMB), jnp.float32),
            jax.ShapeDtypeStruct((1, 1), jnp.float32),
        ),
        out_specs=(
            pl.BlockSpec(memory_space=pltpu.VMEM),
            pl.BlockSpec(memory_space=pltpu.SMEM),
        ),
    )(pooled, W1, b1, W2p, b2p, labels2d)


def kernel(input_ids, labels, emb_table, W1, b1, W2, b2):
    ids2 = input_ids.astype(jnp.int32).reshape(B * 2, CHUNK)
    pooled = _sc_pool(emb_table, ids2)
    W2p = jnp.zeros((HIDDEN, EMB), jnp.float32).at[:, :NUM_CLASSES].set(W2)
    b2p = jnp.zeros((1, EMB), jnp.float32).at[0, :NUM_CLASSES].set(b2)
    logits_pad, loss = _tc_mlp(pooled, W1, b1.reshape(1, HIDDEN), W2p, b2p,
                               labels.astype(jnp.int32).reshape(B, 1))
    return (logits_pad[:, :NUM_CLASSES], loss[0, 0])
